# baseline (speedup 1.0000x reference)
import functools

import jax
import jax.numpy as jnp
from jax import lax
from jax.experimental import pallas as pl
from jax.experimental.pallas import tpu as pltpu
from jax.experimental.pallas import tpu_sc as plsc

NUM_EMB = 1000000
SUB = 32
BATCH = 16384
NC, NS = 2, 16
NW = NC * NS
BPW = BATCH // NW
CHUNK = 128
NCH = BPW // CHUNK
HALF = BPW // 2

_mesh = plsc.VectorSubcoreMesh(core_axis_name="c", subcore_axis_name="s")


@functools.partial(
    pl.kernel,
    mesh=_mesh,
    compiler_params=pltpu.CompilerParams(use_tc_tiling_on_sc=False),
    out_type=jax.ShapeDtypeStruct((BATCH, 2 * SUB), jnp.float32),
    scratch_types=[
        pltpu.VMEM((NCH, CHUNK), jnp.int32),
        pltpu.VMEM((NCH, CHUNK), jnp.int32),
        pltpu.VMEM((HALF, SUB), jnp.float32),
        pltpu.VMEM((HALF, SUB), jnp.float32),
        pltpu.VMEM((HALF, 2 * SUB), jnp.float32),
        pltpu.SemaphoreType.DMA,
    ],
)
def _hash_embed(idx0_hbm, idx1_hbm, t0_hbm, t1_hbm, out_hbm,
                idx0_v, idx1_v, rows0_v, rows1_v, out_v, sem):
    wid = lax.axis_index("s") * NC + lax.axis_index("c")
    base = wid * BPW
    pltpu.sync_copy(idx0_hbm.at[wid], idx0_v)
    pltpu.sync_copy(idx1_hbm.at[wid], idx1_v)
    for half in range(2):
        copies = []
        for jj in range(HALF // 16):
            j = half * (HALF // CHUNK) + jj // (CHUNK // 16)
            k = jj % (CHUNK // 16)
            iv0 = idx0_v[j, pl.ds(k * 16, 16)]
            iv1 = idx1_v[j, pl.ds(k * 16, 16)]
            copies.append(pltpu.async_copy(
                t0_hbm.at[iv0], rows0_v.at[pl.ds(jj * 16, 16)], sem))
            copies.append(pltpu.async_copy(
                t1_hbm.at[iv1], rows1_v.at[pl.ds(jj * 16, 16)], sem))
        for c in copies:
            c.wait()

        @pl.loop(0, HALF)
        def _interleave(r):
            out_v[r, pl.ds(0, 16)] = rows0_v[r, pl.ds(0, 16)]
            out_v[r, pl.ds(16, 16)] = rows0_v[r, pl.ds(16, 16)]
            out_v[r, pl.ds(32, 16)] = rows1_v[r, pl.ds(0, 16)]
            out_v[r, pl.ds(48, 16)] = rows1_v[r, pl.ds(16, 16)]

        pltpu.sync_copy(out_v, out_hbm.at[pl.ds(base + half * HALF, HALF)])


def kernel(indices, table0, table1):
    idx = indices.astype(jnp.int32)
    idx0 = idx[0].reshape(NW, NCH, CHUNK)
    idx1 = idx[1].reshape(NW, NCH, CHUNK)
    return _hash_embed(idx0, idx1, table0, table1)
